# baseline (speedup 1.0000x reference)
import functools

import jax
import jax.numpy as jnp
from jax import lax
from jax.experimental import pallas as pl
from jax.experimental.pallas import tpu as pltpu
from jax.experimental.pallas import tpu_sc as plsc

B, H, W = 8, 512, 512
GH, GW, GD = 16, 16, 8
NC = 12
ZC = GD * NC
N_OUT = 3

ROWS_PW = (B * H) // 32
RH = 16
CW = 256
NG = CW // 16
XHALF = CW * ZC
GROW = GW * ZC


def _fsplat(x):
    return jnp.full((16,), x, dtype=jnp.float32)


def _isplat(x):
    return jnp.full((16,), x, dtype=jnp.int32)


def _body(grid_hbm, guide_hbm, inp_hbm, out_hbm,
          grows_v, x_v, guide_v, inp_v, out_v, sem):
    wid = lax.axis_index("s") * 2 + lax.axis_index("c")
    b = wid // 4
    row_base = (wid % 4) * ROWS_PW

    lane = lax.iota(jnp.int32, 16)
    lane3 = lane * 3
    laneZC = lane * ZC

    def chunk(hb, _):
        r0 = row_base + hb * RH
        m = r0 // RH
        jy0 = jnp.clip((m + 1) // 2 - 1, 0, GH - 1)
        jy1 = jnp.clip((m + 1) // 2, 0, GH - 1)

        cg0 = pltpu.async_copy(grid_hbm.at[b, jy0],
                               grows_v.at[pl.ds(0, GROW)], sem)
        cg1 = pltpu.async_copy(grid_hbm.at[b, jy1],
                               grows_v.at[pl.ds(GROW, GROW)], sem)
        cgu = pltpu.async_copy(guide_hbm.at[b, pl.ds(r0 * W, RH * W)],
                               guide_v, sem)
        cin = pltpu.async_copy(inp_hbm.at[b, pl.ds(r0 * W * 3, RH * W * 3)],
                               inp_v, sem)
        cg0.wait()
        cg1.wait()

        def do_half(ch):
            c0 = ch * CW

            def stage_w(w, _):
                wg = c0 + w
                x0 = (wg + 16) // 32 - 1
                x0c = jnp.clip(x0, 0, GW - 1)
                x1c = jnp.clip(x0 + 1, 0, GW - 1)
                wx = (_fsplat(wg.astype(jnp.float32)) + 0.5) * (1.0 / 32.0) \
                    - 0.5 - _fsplat(x0.astype(jnp.float32))
                for y01 in range(2):
                    gb = y01 * GROW
                    for k in range(ZC // 16):
                        g0 = grows_v[pl.ds(gb + x0c * ZC + k * 16, 16)]
                        g1 = grows_v[pl.ds(gb + x1c * ZC + k * 16, 16)]
                        x_v[pl.ds(y01 * XHALF + w * ZC + k * 16, 16)] = \
                            g0 + wx * (g1 - g0)
                return 0

            lax.fori_loop(0, CW, stage_w, 0)

            def row_body(r, _):
                hrow = r0 + r
                gy = (_fsplat(hrow.astype(jnp.float32)) + 0.5) \
                    * (1.0 / 32.0) - 0.5
                a1 = gy - ((gy + 1.0).astype(jnp.int32)
                           .astype(jnp.float32) - 1.0)
                a0 = 1.0 - a1
                gbase = r * W + c0
                ibase = r * W * 3 + c0 * 3

                def grp_body(g, _):
                    gv = guide_v[pl.ds(gbase + g * 16, 16)]
                    gz = jnp.clip(gv * GD - 0.5, 0.0, GD - 1.0)
                    z0 = gz.astype(jnp.int32)
                    f = gz - z0.astype(jnp.float32)
                    z1 = jnp.minimum(z0 + 1, GD - 1)
                    base = (g * (16 * ZC)) + laneZC
                    i0 = base + z0 * NC
                    i1 = base + z1 * NC
                    i0p = i0 + XHALF
                    i1p = i1 + XHALF

                    ib = (ibase + g * 48) + lane3
                    aug = [plsc.load_gather(inp_v, [ib + i])
                           for i in range(3)]

                    for o in range(N_OUT):
                        cb = 4 * o
                        acc = None
                        for i in range(4):
                            v00 = plsc.load_gather(x_v, [i0 + (cb + i)])
                            v01 = plsc.load_gather(x_v, [i1 + (cb + i)])
                            v10 = plsc.load_gather(x_v, [i0p + (cb + i)])
                            v11 = plsc.load_gather(x_v, [i1p + (cb + i)])
                            cz = a0 * (v00 + f * (v01 - v00)) \
                                + a1 * (v10 + f * (v11 - v10))
                            if i == 3:
                                acc = acc + cz
                            else:
                                t2 = cz * aug[i]
                                acc = t2 if acc is None else acc + t2
                        plsc.store_scatter(out_v, [ib + o], acc)
                    return 0

                lax.fori_loop(0, NG, grp_body, 0)
                return 0

            lax.fori_loop(0, RH, row_body, 0)

        cgu.wait()
        cin.wait()
        do_half(0)
        do_half(1)
        pltpu.sync_copy(out_v, out_hbm.at[b, pl.ds(r0 * W * 3, RH * W * 3)])
        return 0

    lax.fori_loop(0, ROWS_PW // RH, chunk, 0)


@jax.jit
def _run(grid, guide, inp):
    grid_r = grid.reshape(B, GH, GW * ZC)
    guide_f = guide.reshape(B, H * W)
    inp_f = inp.reshape(B, H * W * 3)
    mesh = plsc.VectorSubcoreMesh(core_axis_name="c", subcore_axis_name="s")
    kfn = functools.partial(
        pl.kernel,
        out_type=jax.ShapeDtypeStruct((B, H * W * 3), jnp.float32),
        mesh=mesh,
        compiler_params=pltpu.CompilerParams(
            needs_layout_passes=False, use_tc_tiling_on_sc=False),
        scratch_types=[
            pltpu.VMEM((2 * GROW,), jnp.float32),
            pltpu.VMEM((2 * XHALF,), jnp.float32),
            pltpu.VMEM((RH * W,), jnp.float32),
            pltpu.VMEM((RH * W * 3,), jnp.float32),
            pltpu.VMEM((RH * W * 3,), jnp.float32),
            pltpu.SemaphoreType.DMA,
        ],
    )(_body)
    return kfn(grid_r, guide_f, inp_f).reshape(B, H, W, N_OUT)


def kernel(bilateral_grid, guide, input):
    return _run(bilateral_grid, guide, input)


# speedup vs baseline: 1.0001x; 1.0001x over previous
import functools

import jax
import jax.numpy as jnp
from jax import lax
from jax.experimental import pallas as pl
from jax.experimental.pallas import tpu as pltpu
from jax.experimental.pallas import tpu_sc as plsc

B, H, W = 8, 512, 512
GH, GW, GD = 16, 16, 8
NC = 12
ZC = GD * NC
N_OUT = 3

ROWS_PW = (B * H) // 32
RH = 16
CW = 256
NG = CW // 16
XHALF = CW * ZC
GROW = GW * ZC


def _fsplat(x):
    return jnp.full((16,), x, dtype=jnp.float32)


def _isplat(x):
    return jnp.full((16,), x, dtype=jnp.int32)


def _body(grid_hbm, guide_hbm, inp_hbm, out_hbm,
          grows_v, x_v, guide_v, inp_v, out_v, sem):
    wid = lax.axis_index("s") * 2 + lax.axis_index("c")
    b = wid // 4
    row_base = (wid % 4) * ROWS_PW

    lane = lax.iota(jnp.int32, 16)
    lane3 = lane * 3
    laneZC = lane * ZC

    def chunk(hb, _):
        r0 = row_base + hb * RH
        m = r0 // RH
        jy0 = jnp.clip((m + 1) // 2 - 1, 0, GH - 1)
        jy1 = jnp.clip((m + 1) // 2, 0, GH - 1)

        cg0 = pltpu.async_copy(grid_hbm.at[b, jy0],
                               grows_v.at[pl.ds(0, GROW)], sem)
        cg1 = pltpu.async_copy(grid_hbm.at[b, jy1],
                               grows_v.at[pl.ds(GROW, GROW)], sem)
        cgu = pltpu.async_copy(guide_hbm.at[b, pl.ds(r0 * W, RH * W)],
                               guide_v, sem)
        cin = pltpu.async_copy(inp_hbm.at[b, pl.ds(r0 * W * 3, RH * W * 3)],
                               inp_v, sem)
        cg0.wait()
        cg1.wait()

        def do_half(ch, _):
            c0 = ch * CW

            def stage_w(w, _):
                wg = c0 + w
                x0 = (wg + 16) // 32 - 1
                x0c = jnp.clip(x0, 0, GW - 1)
                x1c = jnp.clip(x0 + 1, 0, GW - 1)
                wx = (_fsplat(wg.astype(jnp.float32)) + 0.5) * (1.0 / 32.0) \
                    - 0.5 - _fsplat(x0.astype(jnp.float32))
                for y01 in range(2):
                    gb = y01 * GROW
                    for k in range(ZC // 16):
                        g0 = grows_v[pl.ds(gb + x0c * ZC + k * 16, 16)]
                        g1 = grows_v[pl.ds(gb + x1c * ZC + k * 16, 16)]
                        x_v[pl.ds(y01 * XHALF + w * ZC + k * 16, 16)] = \
                            g0 + wx * (g1 - g0)
                return 0

            lax.fori_loop(0, CW, stage_w, 0)

            def row_body(r, _):
                hrow = r0 + r
                gy = (_fsplat(hrow.astype(jnp.float32)) + 0.5) \
                    * (1.0 / 32.0) - 0.5
                a1 = gy - ((gy + 1.0).astype(jnp.int32)
                           .astype(jnp.float32) - 1.0)
                a0 = 1.0 - a1
                gbase = r * W + c0
                ibase = r * W * 3 + c0 * 3

                def grp_body(g, _):
                    gv = guide_v[pl.ds(gbase + g * 16, 16)]
                    gz = jnp.clip(gv * GD - 0.5, 0.0, GD - 1.0)
                    z0 = gz.astype(jnp.int32)
                    f = gz - z0.astype(jnp.float32)
                    z1 = jnp.minimum(z0 + 1, GD - 1)
                    base = (g * (16 * ZC)) + laneZC
                    i0 = base + z0 * NC
                    i1 = base + z1 * NC
                    i0p = i0 + XHALF
                    i1p = i1 + XHALF

                    ib = (ibase + g * 48) + lane3
                    aug = [plsc.load_gather(inp_v, [ib + i])
                           for i in range(3)]

                    for o in range(N_OUT):
                        cb = 4 * o
                        acc = None
                        for i in range(4):
                            v00 = plsc.load_gather(x_v, [i0 + (cb + i)])
                            v01 = plsc.load_gather(x_v, [i1 + (cb + i)])
                            v10 = plsc.load_gather(x_v, [i0p + (cb + i)])
                            v11 = plsc.load_gather(x_v, [i1p + (cb + i)])
                            cz = a0 * (v00 + f * (v01 - v00)) \
                                + a1 * (v10 + f * (v11 - v10))
                            if i == 3:
                                acc = acc + cz
                            else:
                                t2 = cz * aug[i]
                                acc = t2 if acc is None else acc + t2
                        plsc.store_scatter(out_v, [ib + o], acc)
                    return 0

                lax.fori_loop(0, NG, grp_body, 0)
                return 0

            lax.fori_loop(0, RH, row_body, 0)
            return 0

        cgu.wait()
        cin.wait()
        lax.fori_loop(0, 2, do_half, 0)
        pltpu.sync_copy(out_v, out_hbm.at[b, pl.ds(r0 * W * 3, RH * W * 3)])
        return 0

    lax.fori_loop(0, ROWS_PW // RH, chunk, 0)


@jax.jit
def _run(grid, guide, inp):
    grid_r = grid.reshape(B, GH, GW * ZC)
    guide_f = guide.reshape(B, H * W)
    inp_f = inp.reshape(B, H * W * 3)
    mesh = plsc.VectorSubcoreMesh(core_axis_name="c", subcore_axis_name="s")
    kfn = functools.partial(
        pl.kernel,
        out_type=jax.ShapeDtypeStruct((B, H * W * 3), jnp.float32),
        mesh=mesh,
        compiler_params=pltpu.CompilerParams(
            needs_layout_passes=False, use_tc_tiling_on_sc=False),
        scratch_types=[
            pltpu.VMEM((2 * GROW,), jnp.float32),
            pltpu.VMEM((2 * XHALF,), jnp.float32),
            pltpu.VMEM((RH * W,), jnp.float32),
            pltpu.VMEM((RH * W * 3,), jnp.float32),
            pltpu.VMEM((RH * W * 3,), jnp.float32),
            pltpu.SemaphoreType.DMA,
        ],
    )(_body)
    return kfn(grid_r, guide_f, inp_f).reshape(B, H, W, N_OUT)


def kernel(bilateral_grid, guide, input):
    return _run(bilateral_grid, guide, input)


# speedup vs baseline: 3.0146x; 3.0143x over previous
import functools

import jax
import jax.numpy as jnp
from jax import lax
from jax.experimental import pallas as pl
from jax.experimental.pallas import tpu as pltpu
from jax.experimental.pallas import tpu_sc as plsc

B, H, W = 8, 512, 512
GH, GW, GD = 16, 16, 8
NC = 12
ZC = GD * NC
N_OUT = 3

ROWS_PW = (B * H) // 32
RH = 16
CW = 256
NG = CW // 16
XHALF = CW * ZC
GROW = GW * ZC


def _fsplat(x):
    return jnp.full((16,), x, dtype=jnp.float32)


def _isplat(x):
    return jnp.full((16,), x, dtype=jnp.int32)


def _body(grid_hbm, guide_hbm, inp_hbm, out_hbm,
          grows_v, x_v, guide_v, inp_v, out_v, sem):
    wid = lax.axis_index("s") * 2 + lax.axis_index("c")
    b = wid // 4
    row_base = (wid % 4) * ROWS_PW

    lane = lax.iota(jnp.int32, 16)
    lane3 = lane * 3
    laneZC = lane * ZC

    def chunk(hb, _):
        r0 = row_base + hb * RH
        m = r0 // RH
        jy0 = jnp.clip((m + 1) // 2 - 1, 0, GH - 1)
        jy1 = jnp.clip((m + 1) // 2, 0, GH - 1)

        cg0 = pltpu.async_copy(grid_hbm.at[b, jy0],
                               grows_v.at[pl.ds(0, GROW)], sem)
        cg1 = pltpu.async_copy(grid_hbm.at[b, jy1],
                               grows_v.at[pl.ds(GROW, GROW)], sem)
        cgu = pltpu.async_copy(guide_hbm.at[b, pl.ds(r0, RH)],
                               guide_v, sem)
        cin = pltpu.async_copy(inp_hbm.at[b, pl.ds(r0, RH)],
                               inp_v, sem)
        cg0.wait()
        cg1.wait()

        def do_half(ch, _):
            c0 = ch * CW

            def stage_w(w, _):
                wg = c0 + w
                x0 = (wg + 16) // 32 - 1
                x0c = jnp.clip(x0, 0, GW - 1)
                x1c = jnp.clip(x0 + 1, 0, GW - 1)
                wx = (_fsplat(wg.astype(jnp.float32)) + 0.5) * (1.0 / 32.0) \
                    - 0.5 - _fsplat(x0.astype(jnp.float32))
                for y01 in range(2):
                    gb = y01 * GROW
                    for k in range(ZC // 16):
                        g0 = grows_v[pl.ds(gb + x0c * ZC + k * 16, 16)]
                        g1 = grows_v[pl.ds(gb + x1c * ZC + k * 16, 16)]
                        x_v[pl.ds(y01 * XHALF + w * ZC + k * 16, 16)] = \
                            g0 + wx * (g1 - g0)
                return 0

            lax.fori_loop(0, CW, stage_w, 0)

            def row_body(r, _):
                hrow = r0 + r
                gy = (_fsplat(hrow.astype(jnp.float32)) + 0.5) \
                    * (1.0 / 32.0) - 0.5
                a1 = gy - ((gy + 1.0).astype(jnp.int32)
                           .astype(jnp.float32) - 1.0)
                a0 = 1.0 - a1
                gbase = c0
                ibase = c0 * 3
                rsp = _isplat(r)

                def grp_body(g, _):
                    gv = guide_v[r, pl.ds(gbase + g * 16, 16)]
                    gz = jnp.clip(gv * GD - 0.5, 0.0, GD - 1.0)
                    z0 = gz.astype(jnp.int32)
                    f = gz - z0.astype(jnp.float32)
                    z1 = jnp.minimum(z0 + 1, GD - 1)
                    base = (g * (16 * ZC)) + laneZC
                    i0 = base + z0 * NC
                    i1 = base + z1 * NC
                    i0p = i0 + XHALF
                    i1p = i1 + XHALF

                    ib = (ibase + g * 48) + lane3
                    aug = [plsc.load_gather(inp_v, [rsp, ib + i])
                           for i in range(3)]

                    for o in range(N_OUT):
                        cb = 4 * o
                        acc = None
                        for i in range(4):
                            v00 = plsc.load_gather(x_v, [i0 + (cb + i)])
                            v01 = plsc.load_gather(x_v, [i1 + (cb + i)])
                            v10 = plsc.load_gather(x_v, [i0p + (cb + i)])
                            v11 = plsc.load_gather(x_v, [i1p + (cb + i)])
                            cz = a0 * (v00 + f * (v01 - v00)) \
                                + a1 * (v10 + f * (v11 - v10))
                            if i == 3:
                                acc = acc + cz
                            else:
                                t2 = cz * aug[i]
                                acc = t2 if acc is None else acc + t2
                        plsc.store_scatter(out_v, [rsp, ib + o], acc)
                    return 0

                lax.fori_loop(0, NG, grp_body, 0)
                return 0

            lax.fori_loop(0, RH, row_body, 0)
            return 0

        cgu.wait()
        cin.wait()
        lax.fori_loop(0, 2, do_half, 0)
        pltpu.sync_copy(out_v, out_hbm.at[b, pl.ds(r0, RH)])
        return 0

    lax.fori_loop(0, ROWS_PW // RH, chunk, 0)


@jax.jit
def _run(grid, guide, inp):
    grid_r = grid.reshape(B, GH, GW * ZC)
    inp_f = inp.reshape(B, H, W * 3)
    mesh = plsc.VectorSubcoreMesh(core_axis_name="c", subcore_axis_name="s")
    kfn = functools.partial(
        pl.kernel,
        out_type=jax.ShapeDtypeStruct((B, H, W * 3), jnp.float32),
        mesh=mesh,
        compiler_params=pltpu.CompilerParams(
            needs_layout_passes=False, use_tc_tiling_on_sc=False),
        scratch_types=[
            pltpu.VMEM((2 * GROW,), jnp.float32),
            pltpu.VMEM((2 * XHALF,), jnp.float32),
            pltpu.VMEM((RH, W), jnp.float32),
            pltpu.VMEM((RH, W * 3), jnp.float32),
            pltpu.VMEM((RH, W * 3), jnp.float32),
            pltpu.SemaphoreType.DMA,
        ],
    )(_body)
    return kfn(grid_r, guide, inp_f).reshape(B, H, W, N_OUT)


def kernel(bilateral_grid, guide, input):
    return _run(bilateral_grid, guide, input)


# speedup vs baseline: 3.4434x; 1.1422x over previous
import functools

import jax
import jax.numpy as jnp
from jax import lax
from jax.experimental import pallas as pl
from jax.experimental.pallas import tpu as pltpu
from jax.experimental.pallas import tpu_sc as plsc

B, H, W = 8, 512, 512
GH, GW, GD = 16, 16, 8
NC = 12
ZC = GD * NC
N_OUT = 3

ROWS_PW = (B * H) // 32
RH = 16
CW = 256
NG = CW // 16
XHALF = CW * ZC
GROW = GW * ZC


def _fsplat(x):
    return jnp.full((16,), x, dtype=jnp.float32)


def _isplat(x):
    return jnp.full((16,), x, dtype=jnp.int32)


def _body(grid_hbm, guide_hbm, inp_hbm, out_hbm,
          grows_v, x_v, guide_v, inp_v, out_v, sem):
    wid = lax.axis_index("s") * 2 + lax.axis_index("c")
    b = wid // 4
    row_base = (wid % 4) * ROWS_PW

    lane = lax.iota(jnp.int32, 16)
    lane3 = lane * 3
    laneZC = lane * ZC

    def chunk(hb, _):
        r0 = row_base + hb * RH
        m = r0 // RH
        jy0 = jnp.clip((m + 1) // 2 - 1, 0, GH - 1)
        jy1 = jnp.clip((m + 1) // 2, 0, GH - 1)

        cg0 = pltpu.async_copy(grid_hbm.at[b, jy0],
                               grows_v.at[pl.ds(0, GROW)], sem)
        cg1 = pltpu.async_copy(grid_hbm.at[b, jy1],
                               grows_v.at[pl.ds(GROW, GROW)], sem)
        cgu = pltpu.async_copy(guide_hbm.at[b, pl.ds(r0, RH)],
                               guide_v, sem)
        cin = pltpu.async_copy(inp_hbm.at[b, pl.ds(r0, RH)],
                               inp_v, sem)
        cg0.wait()
        cg1.wait()

        def do_half(ch, _):
            c0 = ch * CW

            def stage_w(w, _):
                wg = c0 + w
                x0 = (wg + 16) // 32 - 1
                x0c = jnp.clip(x0, 0, GW - 1)
                x1c = jnp.clip(x0 + 1, 0, GW - 1)
                wx = (_fsplat(wg.astype(jnp.float32)) + 0.5) * (1.0 / 32.0) \
                    - 0.5 - _fsplat(x0.astype(jnp.float32))
                for y01 in range(2):
                    gb = y01 * GROW
                    for k in range(ZC // 16):
                        g0 = grows_v[pl.ds(gb + x0c * ZC + k * 16, 16)]
                        g1 = grows_v[pl.ds(gb + x1c * ZC + k * 16, 16)]
                        x_v[pl.ds(y01 * XHALF + w * ZC + k * 16, 16)] = \
                            g0 + wx * (g1 - g0)
                return 0

            lax.fori_loop(0, CW, stage_w, 0)

            def row_body(r, _):
                hrow = r0 + r
                gy = (_fsplat(hrow.astype(jnp.float32)) + 0.5) \
                    * (1.0 / 32.0) - 0.5
                a1 = gy - ((gy + 1.0).astype(jnp.int32)
                           .astype(jnp.float32) - 1.0)
                a0 = 1.0 - a1
                gbase = c0
                ibase = c0 * 3
                rsp = _isplat(r)

                @plsc.parallel_loop(0, NG, unroll=1)
                def grp_body(g):
                    gv = guide_v[r, pl.ds(gbase + g * 16, 16)]
                    gz = jnp.clip(gv * GD - 0.5, 0.0, GD - 1.0)
                    z0 = gz.astype(jnp.int32)
                    f = gz - z0.astype(jnp.float32)
                    z1 = jnp.minimum(z0 + 1, GD - 1)
                    base = (g * (16 * ZC)) + laneZC
                    i0 = base + z0 * NC
                    i1 = base + z1 * NC
                    i0p = i0 + XHALF
                    i1p = i1 + XHALF

                    ib = (ibase + g * 48) + lane3
                    aug = [plsc.load_gather(inp_v, [rsp, ib + i])
                           for i in range(3)]

                    for o in range(N_OUT):
                        cb = 4 * o
                        acc = None
                        for i in range(4):
                            v00 = plsc.load_gather(x_v, [i0 + (cb + i)])
                            v01 = plsc.load_gather(x_v, [i1 + (cb + i)])
                            v10 = plsc.load_gather(x_v, [i0p + (cb + i)])
                            v11 = plsc.load_gather(x_v, [i1p + (cb + i)])
                            cz = a0 * (v00 + f * (v01 - v00)) \
                                + a1 * (v10 + f * (v11 - v10))
                            if i == 3:
                                acc = acc + cz
                            else:
                                t2 = cz * aug[i]
                                acc = t2 if acc is None else acc + t2
                        plsc.store_scatter(out_v, [rsp, ib + o], acc)

                return 0

            lax.fori_loop(0, RH, row_body, 0)
            return 0

        cgu.wait()
        cin.wait()
        lax.fori_loop(0, 2, do_half, 0)
        pltpu.sync_copy(out_v, out_hbm.at[b, pl.ds(r0, RH)])
        return 0

    lax.fori_loop(0, ROWS_PW // RH, chunk, 0)


@jax.jit
def _run(grid, guide, inp):
    grid_r = grid.reshape(B, GH, GW * ZC)
    inp_f = inp.reshape(B, H, W * 3)
    mesh = plsc.VectorSubcoreMesh(core_axis_name="c", subcore_axis_name="s")
    kfn = functools.partial(
        pl.kernel,
        out_type=jax.ShapeDtypeStruct((B, H, W * 3), jnp.float32),
        mesh=mesh,
        compiler_params=pltpu.CompilerParams(
            needs_layout_passes=False, use_tc_tiling_on_sc=False),
        scratch_types=[
            pltpu.VMEM((2 * GROW,), jnp.float32),
            pltpu.VMEM((2 * XHALF,), jnp.float32),
            pltpu.VMEM((RH, W), jnp.float32),
            pltpu.VMEM((RH, W * 3), jnp.float32),
            pltpu.VMEM((RH, W * 3), jnp.float32),
            pltpu.SemaphoreType.DMA,
        ],
    )(_body)
    return kfn(grid_r, guide, inp_f).reshape(B, H, W, N_OUT)


def kernel(bilateral_grid, guide, input):
    return _run(bilateral_grid, guide, input)
